# baseline (speedup 1.0000x reference)
import dataclasses
import functools

import jax
import jax.numpy as jnp
from jax.experimental import pallas as pl
from jax.experimental.pallas import tpu as pltpu
from jax.experimental.pallas import tpu_sc as plsc

EPS = 1e-6
NB = 1024
PREP_CH = 4096
GW = 2048
PAD = 8


def _prep_body(idx_ref, msk_ref, sel_ref, *, n, npad):
    b = pl.program_id(0)
    sel_ref[...] = (jnp.where(msk_ref[...] != 0, idx_ref[...], n)
                    + b * npad) * 2


def _sc_gather_body(x_hbm, s_hbm, o0_hbm, o1_hbm, tab, sem, *, bc, k, n):
    pltpu.async_copy(x_hbm, tab, sem).wait()

    def body(s_vmem, o0_vmem, o1_vmem):
        @pl.loop(0, GW, step=16)
        def _(c):
            s2 = s_vmem[0, 0, pl.ds(c, 16)]
            o0_vmem[0, 0, pl.ds(c, 16)] = plsc.load_gather(tab, [s2])
            o1_vmem[0, 0, pl.ds(c, 16)] = plsc.load_gather(tab, [s2 + 1])

    npc = n // GW
    idxmap = lambda i: (i // (k * npc), (i // npc) % k, i % npc)
    pltpu.emit_pipeline(
        body,
        grid=(bc * k * npc,),
        in_specs=[pl.BlockSpec((1, 1, GW), idxmap)],
        out_specs=[pl.BlockSpec((1, 1, GW), idxmap),
                   pl.BlockSpec((1, 1, GW), idxmap)],
        core_axis_name=("core", "subcore"),
        dimension_semantics=(pltpu.PARALLEL,),
    )(s_hbm, o0_hbm, o1_hbm)


def _sc_gather(x_flat, sel3, bc, k, n):
    mesh = plsc.VectorSubcoreMesh(core_axis_name="core",
                                  subcore_axis_name="subcore")
    cp = pltpu.CompilerParams()
    if "needs_layout_passes" in pltpu.CompilerParams.__dataclass_fields__:
        cp = dataclasses.replace(cp, needs_layout_passes=False)
    gk = pl.kernel(
        functools.partial(_sc_gather_body, bc=bc, k=k, n=n),
        out_type=[jax.ShapeDtypeStruct((bc, k, n), jnp.float32),
                  jax.ShapeDtypeStruct((bc, k, n), jnp.float32)],
        mesh=mesh,
        scratch_types=[pltpu.VMEM(x_flat.shape, jnp.float32),
                       pltpu.SemaphoreType.DMA],
        compiler_params=cp,
    )
    return gk(x_flat, sel3)


def _vn_nonlin(p0, p1, q0, q1):
    dot = p0 * q0 + p1 * q1
    dd = q0 * q0 + q1 * q1
    r = jnp.minimum(dot, 0.0) / (dd + EPS)
    f0 = p0 - r * q0
    f1 = p1 - r * q1
    return f0, f1


def _dense_body(g0_ref, g1_ref, x0_ref, x1_ref, w0_ref, d0_ref, dp_ref,
                w1_ref, d1_ref, w2_ref, d2_ref, w3_ref, d3_ref,
                wo_ref, do_ref, out_ref):
    k = g0_ref.shape[1]
    lanes = g0_ref.shape[2]
    kl = k * lanes
    c0 = w0_ref.shape[0]

    xn0 = g0_ref[...].reshape(1, kl)
    xn1 = g1_ref[...].reshape(1, kl)
    xs0 = jnp.broadcast_to(x0_ref[...].reshape(1, lanes),
                           (k, lanes)).reshape(1, kl)
    xs1 = jnp.broadcast_to(x1_ref[...].reshape(1, lanes),
                           (k, lanes)).reshape(1, kl)

    wd0 = jnp.concatenate([w0_ref[...], d0_ref[...]], axis=0)
    pq0 = jnp.dot(wd0, jnp.concatenate([xn0, xs0], axis=0),
                  preferred_element_type=jnp.float32)
    pq1 = jnp.dot(wd0, jnp.concatenate([xn1, xs1], axis=0),
                  preferred_element_type=jnp.float32)
    f0, f1 = _vn_nonlin(pq0[:c0], pq1[:c0],
                        pq0[c0:], pq1[c0:])

    dpw = dp_ref[...]
    dp0 = jnp.dot(dpw, f0, preferred_element_type=jnp.float32)
    dp1 = jnp.dot(dpw, f1, preferred_element_type=jnp.float32)
    s = f0 * dp0 + f1 * dp1

    def kslc(a, kk):
        return a[:, kk * lanes:(kk + 1) * lanes]

    m = kslc(s, 0)
    for kk in range(1, k):
        m = jnp.maximum(m, kslc(s, kk))
    p0 = kslc(f0, k - 1)
    p1 = kslc(f1, k - 1)
    for kk in range(k - 2, -1, -1):
        hit = kslc(s, kk) == m
        p0 = jnp.where(hit, kslc(f0, kk), p0)
        p1 = jnp.where(hit, kslc(f1, kk), p1)
    f0, f1 = p0, p1

    for wr, dr in ((w1_ref, d1_ref), (w2_ref, d2_ref),
                   (w3_ref, d3_ref), (wo_ref, do_ref)):
        wd = jnp.concatenate([wr[...], dr[...]], axis=0)
        co = wr.shape[0]
        pq0 = jnp.dot(wd, f0, preferred_element_type=jnp.float32)
        pq1 = jnp.dot(wd, f1, preferred_element_type=jnp.float32)
        f0, f1 = _vn_nonlin(pq0[:co], pq1[:co], pq0[co:], pq1[co:])

    out_ref[0, :, 0, :] = f0
    out_ref[0, :, 1, :] = f1


CHUNKS = 8


def kernel(x, neigh_ped_mask, near_ped_idx,
           W0, D0, Dpool, W1, D1, W2, D2, W3, D3, Wout, Dout):
    B, N, Dv = x.shape
    K = near_ped_idx.shape[-1]
    npad = N + PAD
    bc = B // CHUNKS
    totc = bc * K * N
    out_c = Wout.shape[0]

    idx_t = jnp.swapaxes(near_ped_idx.astype(jnp.int32), 1, 2)
    msk_t = jnp.swapaxes(neigh_ped_mask.astype(jnp.int32), 1, 2)
    x_ext = jnp.concatenate([x, jnp.zeros((B, PAD, Dv), x.dtype)], axis=1)

    wspec = lambda w: pl.BlockSpec(w.shape, lambda b, j: (0, 0))
    gathered = []
    for c in range(CHUNKS):
        sl = slice(c * bc, (c + 1) * bc)
        sel = pl.pallas_call(
            functools.partial(_prep_body, n=N, npad=npad),
            grid=(bc, N // PREP_CH),
            in_specs=[pl.BlockSpec((1, K, PREP_CH), lambda b, j: (b, 0, j)),
                      pl.BlockSpec((1, K, PREP_CH), lambda b, j: (b, 0, j))],
            out_specs=pl.BlockSpec((1, K, PREP_CH), lambda b, j: (b, 0, j)),
            out_shape=jax.ShapeDtypeStruct((bc, K, N), jnp.int32),
        )(idx_t[sl], msk_t[sl])

        x_flat = x_ext[sl].reshape(bc * npad * Dv)
        g0, g1 = _sc_gather(x_flat, sel, bc, K, N)
        gathered.append((g0, g1))

    outs = []
    for c in range(CHUNKS):
        sl = slice(c * bc, (c + 1) * bc)
        g0, g1 = gathered[c]
        x0 = x[sl, :, 0].reshape(bc, 1, N)
        x1 = x[sl, :, 1].reshape(bc, 1, N)

        out = pl.pallas_call(
            _dense_body,
            grid=(bc, N // NB),
            in_specs=[
                pl.BlockSpec((1, K, NB), lambda b, j: (b, 0, j)),
                pl.BlockSpec((1, K, NB), lambda b, j: (b, 0, j)),
                pl.BlockSpec((1, 1, NB), lambda b, j: (b, 0, j)),
                pl.BlockSpec((1, 1, NB), lambda b, j: (b, 0, j)),
                wspec(W0), wspec(D0), wspec(Dpool), wspec(W1), wspec(D1),
                wspec(W2), wspec(D2), wspec(W3), wspec(D3),
                wspec(Wout), wspec(Dout),
            ],
            out_specs=pl.BlockSpec((1, out_c, 2, NB),
                                   lambda b, j: (b, 0, 0, j)),
            out_shape=jax.ShapeDtypeStruct((bc, out_c, 2, N), jnp.float32),
        )(g0, g1, x0, x1, W0, D0, Dpool, W1, D1, W2, D2, W3, D3, Wout, Dout)
        outs.append(out)

    return jnp.concatenate(outs, axis=0)


# speedup vs baseline: 1.0305x; 1.0305x over previous
import dataclasses
import functools

import jax
import jax.numpy as jnp
from jax.experimental import pallas as pl
from jax.experimental.pallas import tpu as pltpu
from jax.experimental.pallas import tpu_sc as plsc

EPS = 1e-6
NB = 1024
PREP_CH = 4096
GW = 2048
PAD = 8


def _prep_body(idx_ref, msk_ref, sel_ref, *, n, npad):
    b = pl.program_id(0)
    sel_ref[...] = (jnp.where(msk_ref[...] != 0, idx_ref[...], n)
                    + b * npad) * 2


def _sc_gather_body(x_hbm, s_hbm, o0_hbm, o1_hbm, tab, sem, *, bc, k, n):
    pltpu.async_copy(x_hbm, tab, sem).wait()

    def body(s_vmem, o0_vmem, o1_vmem):
        @pl.loop(0, GW, step=16)
        def _(c):
            s2 = s_vmem[0, 0, pl.ds(c, 16)]
            o0_vmem[0, 0, pl.ds(c, 16)] = plsc.load_gather(tab, [s2])
            o1_vmem[0, 0, pl.ds(c, 16)] = plsc.load_gather(tab, [s2 + 1])

    npc = n // GW
    idxmap = lambda i: (i // (k * npc), (i // npc) % k, i % npc)
    pltpu.emit_pipeline(
        body,
        grid=(bc * k * npc,),
        in_specs=[pl.BlockSpec((1, 1, GW), idxmap)],
        out_specs=[pl.BlockSpec((1, 1, GW), idxmap),
                   pl.BlockSpec((1, 1, GW), idxmap)],
        core_axis_name=("core", "subcore"),
        dimension_semantics=(pltpu.PARALLEL,),
    )(s_hbm, o0_hbm, o1_hbm)


def _sc_gather(x_flat, sel3, bc, k, n):
    mesh = plsc.VectorSubcoreMesh(core_axis_name="core",
                                  subcore_axis_name="subcore")
    cp = pltpu.CompilerParams()
    if "needs_layout_passes" in pltpu.CompilerParams.__dataclass_fields__:
        cp = dataclasses.replace(cp, needs_layout_passes=False)
    gk = pl.kernel(
        functools.partial(_sc_gather_body, bc=bc, k=k, n=n),
        out_type=[jax.ShapeDtypeStruct((bc, k, n), jnp.float32),
                  jax.ShapeDtypeStruct((bc, k, n), jnp.float32)],
        mesh=mesh,
        scratch_types=[pltpu.VMEM(x_flat.shape, jnp.float32),
                       pltpu.SemaphoreType.DMA],
        compiler_params=cp,
    )
    return gk(x_flat, sel3)


def _vn_nonlin(p0, p1, q0, q1):
    dot = p0 * q0 + p1 * q1
    dd = q0 * q0 + q1 * q1
    r = jnp.minimum(dot, 0.0) / (dd + EPS)
    f0 = p0 - r * q0
    f1 = p1 - r * q1
    return f0, f1


def _dense_body(g0_ref, g1_ref, x0_ref, x1_ref, w0_ref, d0_ref, dp_ref,
                w1_ref, d1_ref, w2_ref, d2_ref, w3_ref, d3_ref,
                wo_ref, do_ref, out_ref):
    k = g0_ref.shape[1]
    lanes = g0_ref.shape[2]
    kl = k * lanes
    c0 = w0_ref.shape[0]

    xn0 = g0_ref[...].reshape(1, kl)
    xn1 = g1_ref[...].reshape(1, kl)
    xs0 = jnp.broadcast_to(x0_ref[...].reshape(1, lanes),
                           (k, lanes)).reshape(1, kl)
    xs1 = jnp.broadcast_to(x1_ref[...].reshape(1, lanes),
                           (k, lanes)).reshape(1, kl)

    wd0 = jnp.concatenate([w0_ref[...], d0_ref[...]], axis=0)
    pq0 = jnp.dot(wd0, jnp.concatenate([xn0, xs0], axis=0),
                  preferred_element_type=jnp.float32)
    pq1 = jnp.dot(wd0, jnp.concatenate([xn1, xs1], axis=0),
                  preferred_element_type=jnp.float32)
    f0, f1 = _vn_nonlin(pq0[:c0], pq1[:c0],
                        pq0[c0:], pq1[c0:])

    dpw = dp_ref[...]
    dp0 = jnp.dot(dpw, f0, preferred_element_type=jnp.float32)
    dp1 = jnp.dot(dpw, f1, preferred_element_type=jnp.float32)
    s = f0 * dp0 + f1 * dp1

    def kslc(a, kk):
        return a[:, kk * lanes:(kk + 1) * lanes]

    m = kslc(s, 0)
    for kk in range(1, k):
        m = jnp.maximum(m, kslc(s, kk))
    p0 = kslc(f0, k - 1)
    p1 = kslc(f1, k - 1)
    for kk in range(k - 2, -1, -1):
        hit = kslc(s, kk) == m
        p0 = jnp.where(hit, kslc(f0, kk), p0)
        p1 = jnp.where(hit, kslc(f1, kk), p1)
    f0, f1 = p0, p1

    for wr, dr in ((w1_ref, d1_ref), (w2_ref, d2_ref),
                   (w3_ref, d3_ref), (wo_ref, do_ref)):
        wd = jnp.concatenate([wr[...], dr[...]], axis=0)
        co = wr.shape[0]
        pq0 = jnp.dot(wd, f0, preferred_element_type=jnp.float32)
        pq1 = jnp.dot(wd, f1, preferred_element_type=jnp.float32)
        f0, f1 = _vn_nonlin(pq0[:co], pq1[:co], pq0[co:], pq1[co:])

    out_ref[0, :, 0, :] = f0
    out_ref[0, :, 1, :] = f1


CHUNKS = 4


def kernel(x, neigh_ped_mask, near_ped_idx,
           W0, D0, Dpool, W1, D1, W2, D2, W3, D3, Wout, Dout):
    B, N, Dv = x.shape
    K = near_ped_idx.shape[-1]
    npad = N + PAD
    bc = B // CHUNKS
    out_c = Wout.shape[0]

    idx_t = jnp.swapaxes(near_ped_idx.astype(jnp.int32), 1, 2)
    msk_t = jnp.swapaxes(neigh_ped_mask.astype(jnp.int32), 1, 2)
    x_ext = jnp.concatenate([x, jnp.zeros((B, PAD, Dv), x.dtype)], axis=1)

    wspec = lambda w: pl.BlockSpec(w.shape, lambda b, j: (0, 0))
    gathered = []
    for c in range(CHUNKS):
        sl = slice(c * bc, (c + 1) * bc)
        sel = pl.pallas_call(
            functools.partial(_prep_body, n=N, npad=npad),
            grid=(bc, N // PREP_CH),
            in_specs=[pl.BlockSpec((1, K, PREP_CH), lambda b, j: (b, 0, j)),
                      pl.BlockSpec((1, K, PREP_CH), lambda b, j: (b, 0, j))],
            out_specs=pl.BlockSpec((1, K, PREP_CH), lambda b, j: (b, 0, j)),
            out_shape=jax.ShapeDtypeStruct((bc, K, N), jnp.int32),
        )(idx_t[sl], msk_t[sl])

        x_flat = x_ext[sl].reshape(bc * npad * Dv)
        g0, g1 = _sc_gather(x_flat, sel, bc, K, N)
        gathered.append((g0, g1))

    outs = []
    for c in range(CHUNKS):
        sl = slice(c * bc, (c + 1) * bc)
        g0, g1 = gathered[c]
        x0 = x[sl, :, 0].reshape(bc, 1, N)
        x1 = x[sl, :, 1].reshape(bc, 1, N)

        out = pl.pallas_call(
            _dense_body,
            grid=(bc, N // NB),
            in_specs=[
                pl.BlockSpec((1, K, NB), lambda b, j: (b, 0, j)),
                pl.BlockSpec((1, K, NB), lambda b, j: (b, 0, j)),
                pl.BlockSpec((1, 1, NB), lambda b, j: (b, 0, j)),
                pl.BlockSpec((1, 1, NB), lambda b, j: (b, 0, j)),
                wspec(W0), wspec(D0), wspec(Dpool), wspec(W1), wspec(D1),
                wspec(W2), wspec(D2), wspec(W3), wspec(D3),
                wspec(Wout), wspec(Dout),
            ],
            out_specs=pl.BlockSpec((1, out_c, 2, NB),
                                   lambda b, j: (b, 0, 0, j)),
            out_shape=jax.ShapeDtypeStruct((bc, out_c, 2, N), jnp.float32),
        )(g0, g1, x0, x1, W0, D0, Dpool, W1, D1, W2, D2, W3, D3, Wout, Dout)
        outs.append(out)

    return jnp.concatenate(outs, axis=0)


# speedup vs baseline: 1.0475x; 1.0165x over previous
import dataclasses
import functools

import jax
import jax.numpy as jnp
from jax.experimental import pallas as pl
from jax.experimental.pallas import tpu as pltpu
from jax.experimental.pallas import tpu_sc as plsc

EPS = 1e-6
NB = 1024
PREP_CH = 4096
GW = 2048
PAD = 8


def _prep_body(idx_ref, msk_ref, sel_ref, *, n, npad):
    b = pl.program_id(0)
    off = b * npad
    sel_ref[0, 0] = (jnp.where(msk_ref[0] != 0, idx_ref[0], n) + off) * 2
    k, pch = idx_ref.shape[1], idx_ref.shape[2]
    nio = jax.lax.broadcasted_iota(jnp.int32, (k, pch), 1)
    sel_ref[0, 1] = (nio + (pl.program_id(1) * pch + off)) * 2


def _sc_gather_body(x_hbm, s_hbm, o0_hbm, o1_hbm, tab, sem, *, bc, k, n):
    pltpu.async_copy(x_hbm, tab, sem).wait()

    def body(s_vmem, o0_vmem, o1_vmem):
        @pl.loop(0, GW, step=16)
        def _(c):
            s2 = s_vmem[0, 0, 0, pl.ds(c, 16)]
            o0_vmem[0, 0, 0, pl.ds(c, 16)] = plsc.load_gather(tab, [s2])
            o1_vmem[0, 0, 0, pl.ds(c, 16)] = plsc.load_gather(tab, [s2 + 1])

    npc = n // GW
    idxmap = lambda i: (i // (2 * k * npc), (i // (k * npc)) % 2,
                        (i // npc) % k, i % npc)
    pltpu.emit_pipeline(
        body,
        grid=(bc * 2 * k * npc,),
        in_specs=[pl.BlockSpec((1, 1, 1, GW), idxmap)],
        out_specs=[pl.BlockSpec((1, 1, 1, GW), idxmap),
                   pl.BlockSpec((1, 1, 1, GW), idxmap)],
        core_axis_name=("core", "subcore"),
        dimension_semantics=(pltpu.PARALLEL,),
    )(s_hbm, o0_hbm, o1_hbm)


def _sc_gather(x_flat, sel3, bc, k, n):
    mesh = plsc.VectorSubcoreMesh(core_axis_name="core",
                                  subcore_axis_name="subcore")
    cp = pltpu.CompilerParams()
    if "needs_layout_passes" in pltpu.CompilerParams.__dataclass_fields__:
        cp = dataclasses.replace(cp, needs_layout_passes=False)
    gk = pl.kernel(
        functools.partial(_sc_gather_body, bc=bc, k=k, n=n),
        out_type=[jax.ShapeDtypeStruct((bc, 2, k, n), jnp.float32),
                  jax.ShapeDtypeStruct((bc, 2, k, n), jnp.float32)],
        mesh=mesh,
        scratch_types=[pltpu.VMEM(x_flat.shape, jnp.float32),
                       pltpu.SemaphoreType.DMA],
        compiler_params=cp,
    )
    return gk(x_flat, sel3)


def _vn_nonlin(p0, p1, q0, q1):
    dot = p0 * q0 + p1 * q1
    dd = q0 * q0 + q1 * q1
    r = jnp.minimum(dot, 0.0) / (dd + EPS)
    f0 = p0 - r * q0
    f1 = p1 - r * q1
    return f0, f1


def _dense_body(g0_ref, g1_ref, w0_ref, d0_ref, dp_ref,
                w1_ref, d1_ref, w2_ref, d2_ref, w3_ref, d3_ref,
                wo_ref, do_ref, out_ref):
    k = g0_ref.shape[2]
    lanes = g0_ref.shape[3]
    kl = k * lanes
    c0 = w0_ref.shape[0]

    x0 = g0_ref[...].reshape(2, kl)
    x1 = g1_ref[...].reshape(2, kl)

    wd0 = jnp.concatenate([w0_ref[...], d0_ref[...]], axis=0)
    pq0 = jnp.dot(wd0, x0, preferred_element_type=jnp.float32)
    pq1 = jnp.dot(wd0, x1, preferred_element_type=jnp.float32)
    f0, f1 = _vn_nonlin(pq0[:c0], pq1[:c0],
                        pq0[c0:], pq1[c0:])

    dpw = dp_ref[...]
    dp0 = jnp.dot(dpw, f0, preferred_element_type=jnp.float32)
    dp1 = jnp.dot(dpw, f1, preferred_element_type=jnp.float32)
    s = f0 * dp0 + f1 * dp1

    def kslc(a, kk):
        return a[:, kk * lanes:(kk + 1) * lanes]

    m = kslc(s, 0)
    for kk in range(1, k):
        m = jnp.maximum(m, kslc(s, kk))
    p0 = kslc(f0, k - 1)
    p1 = kslc(f1, k - 1)
    for kk in range(k - 2, -1, -1):
        hit = kslc(s, kk) == m
        p0 = jnp.where(hit, kslc(f0, kk), p0)
        p1 = jnp.where(hit, kslc(f1, kk), p1)
    f0, f1 = p0, p1

    for wr, dr in ((w1_ref, d1_ref), (w2_ref, d2_ref),
                   (w3_ref, d3_ref), (wo_ref, do_ref)):
        wd = jnp.concatenate([wr[...], dr[...]], axis=0)
        co = wr.shape[0]
        pq0 = jnp.dot(wd, f0, preferred_element_type=jnp.float32)
        pq1 = jnp.dot(wd, f1, preferred_element_type=jnp.float32)
        f0, f1 = _vn_nonlin(pq0[:co], pq1[:co], pq0[co:], pq1[co:])

    out_ref[0, :, 0, :] = f0
    out_ref[0, :, 1, :] = f1


CHUNKS = 4


def kernel(x, neigh_ped_mask, near_ped_idx,
           W0, D0, Dpool, W1, D1, W2, D2, W3, D3, Wout, Dout):
    B, N, Dv = x.shape
    K = near_ped_idx.shape[-1]
    npad = N + PAD
    bc = B // CHUNKS
    out_c = Wout.shape[0]

    idx_t = jnp.swapaxes(near_ped_idx.astype(jnp.int32), 1, 2)
    msk_t = jnp.swapaxes(neigh_ped_mask.astype(jnp.int32), 1, 2)
    x_ext = jnp.concatenate([x, jnp.zeros((B, PAD, Dv), x.dtype)], axis=1)

    wspec = lambda w: pl.BlockSpec(w.shape, lambda b, j: (0, 0))
    gathered = []
    for c in range(CHUNKS):
        sl = slice(c * bc, (c + 1) * bc)
        sel = pl.pallas_call(
            functools.partial(_prep_body, n=N, npad=npad),
            grid=(bc, N // PREP_CH),
            in_specs=[pl.BlockSpec((1, K, PREP_CH), lambda b, j: (b, 0, j)),
                      pl.BlockSpec((1, K, PREP_CH), lambda b, j: (b, 0, j))],
            out_specs=pl.BlockSpec((1, 2, K, PREP_CH),
                                   lambda b, j: (b, 0, 0, j)),
            out_shape=jax.ShapeDtypeStruct((bc, 2, K, N), jnp.int32),
        )(idx_t[sl], msk_t[sl])

        x_flat = x_ext[sl].reshape(bc * npad * Dv)
        g0, g1 = _sc_gather(x_flat, sel, bc, K, N)
        gathered.append((g0, g1))

    outs = []
    for c in range(CHUNKS):
        g0, g1 = gathered[c]

        out = pl.pallas_call(
            _dense_body,
            grid=(bc, N // NB),
            in_specs=[
                pl.BlockSpec((1, 2, K, NB), lambda b, j: (b, 0, 0, j)),
                pl.BlockSpec((1, 2, K, NB), lambda b, j: (b, 0, 0, j)),
                wspec(W0), wspec(D0), wspec(Dpool), wspec(W1), wspec(D1),
                wspec(W2), wspec(D2), wspec(W3), wspec(D3),
                wspec(Wout), wspec(Dout),
            ],
            out_specs=pl.BlockSpec((1, out_c, 2, NB),
                                   lambda b, j: (b, 0, 0, j)),
            out_shape=jax.ShapeDtypeStruct((bc, out_c, 2, N), jnp.float32),
        )(g0, g1, W0, D0, Dpool, W1, D1, W2, D2, W3, D3, Wout, Dout)
        outs.append(out)

    return jnp.concatenate(outs, axis=0)
